# baseline (speedup 1.0000x reference)
import functools

import jax
import jax.numpy as jnp
from jax import lax
from jax.experimental import pallas as pl
from jax.experimental.pallas import tpu as pltpu
from jax.experimental.pallas import tpu_sc as plsc

B = 4096
L = 200
C = 64
NT = 3
NC, NS = 2, 16
NW = NC * NS
BPW = B // NW
LP = 208
SP = 201
GC = 104
G2 = 96
BLK = C * L
SBUF = C * SP + 16
SCALE = 8.0


def _body(p_idx, a_idx, f_idx, p_emb, a_emb, f_emb, out_ref,
          i_s0, i_s1, r_v0, r_v1, r_v2, scat_v, o_v0, o_v1, o_v2,
          is0, is1, gs0, gs1, gs2, os0, os1, os2):
  islab = [i_s0, i_s1]
  rows_v = [r_v0, r_v1, r_v2]
  out_v = [o_v0, o_v1, o_v2]
  isem = [is0, is1]
  gsem = [gs0, gs1, gs2]
  osem = [os0, os1, os2]
  tabs = [p_emb, a_emb, f_emb]
  idx_hbm = [p_idx, a_idx, f_idx]

  wid = lax.axis_index("s") * NC + lax.axis_index("c")
  b0 = wid * BPW
  iota = lax.iota(jnp.int32, 16)
  cvec = [(c0 + iota) * SP for c0 in range(0, C, 16)]

  def fire_idx(par, b):
    for t in range(NT):
      pltpu.async_copy(idx_hbm[t].at[b, pl.ds(0, GC)],
                       islab[par].at[2 * t], isem[par])
      pltpu.async_copy(idx_hbm[t].at[b, pl.ds(G2, GC)],
                       islab[par].at[2 * t + 1], isem[par])

  def wait_idx(par, b):
    for t in range(NT):
      pltpu.make_async_copy(idx_hbm[t].at[b, pl.ds(0, GC)],
                            islab[par].at[2 * t], isem[par]).wait()
      pltpu.make_async_copy(idx_hbm[t].at[b, pl.ds(G2, GC)],
                            islab[par].at[2 * t + 1], isem[par]).wait()

  def fire_gather(t, par):
    pltpu.async_copy(tabs[t].at[islab[par].at[2 * t]],
                     rows_v[t].at[pl.ds(0, GC)], gsem[t])
    pltpu.async_copy(tabs[t].at[islab[par].at[2 * t + 1]],
                     rows_v[t].at[pl.ds(G2, GC)], gsem[t])

  def wait_gather(t, par):
    pltpu.make_async_copy(tabs[t].at[islab[par].at[2 * t]],
                          rows_v[t].at[pl.ds(0, GC)], gsem[t]).wait()
    pltpu.make_async_copy(tabs[t].at[islab[par].at[2 * t + 1]],
                          rows_v[t].at[pl.ds(G2, GC)], gsem[t]).wait()

  def fire_out(t, b):
    return pltpu.async_copy(out_v[t].at[pl.ds(0, BLK)],
                            out_ref.at[pl.ds((b * NT + t) * BLK, BLK)],
                            osem[t])

  def wait_out(t, b):
    pltpu.make_async_copy(out_v[t].at[pl.ds(0, BLK)],
                          out_ref.at[pl.ds((b * NT + t) * BLK, BLK)],
                          osem[t]).wait()

  CHUNKS = list(range(0, L - 16, 16)) + [L - 16]
  iotas = {l0: iota + l0 for l0 in CHUNKS}

  def transpose(t):
    rv = rows_v[t]
    ov = out_v[t]

    CHL = 8

    def scat_l(li, carry):
      l = li * CHL
      vs = []
      for dl in range(CHL):
        row = rv.at[l + dl]
        for j in range(C // 16):
          vs.append(row[pl.ds(16 * j, 16)])
      idxs = []
      for dl in range(CHL):
        for j in range(C // 16):
          idxs.append(cvec[j] + (l + dl))
      for v, ix in zip(vs, idxs):
        plsc.store_scatter(scat_v, [ix], v)
      return carry

    lax.fori_loop(0, L // CHL, scat_l, 0)

    CHC = 2

    def comp_c(ci, carry):
      c = ci * CHC
      vs = []
      for dc in range(CHC):
        base = (c + dc) * SP
        for l0 in CHUNKS:
          vs.append(plsc.load_gather(scat_v, [base + iotas[l0]]))
      k = 0
      for dc in range(CHC):
        for l0 in CHUNKS:
          ov[pl.ds((c + dc) * L + l0, 16)] = vs[k] * SCALE
          k += 1
      return carry

    lax.fori_loop(0, C // CHC, comp_c, 0)

  fire_idx(0, b0)
  wait_idx(0, b0)
  fire_gather(0, 0)

  def j_loop(j, carry):
    b = b0 + 2 * j
    bo = b + 1
    bn = b0 + jnp.minimum(2 * j + 2, BPW - 1)

    for t in range(NT):
      if t == 0:
        fire_idx(1, bo)
      if t < NT - 1:
        fire_gather(t + 1, 0)
      else:
        wait_idx(1, bo)
        fire_gather(0, 1)
      wait_gather(t, 0)

      @pl.when(j > 0)
      def _():
        wait_out(t, b - 1)

      transpose(t)
      fire_out(t, b)

    for t in range(NT):
      if t == 0:
        fire_idx(0, bn)
      if t < NT - 1:
        fire_gather(t + 1, 1)
      else:
        wait_idx(0, bn)
        fire_gather(0, 0)
      wait_gather(t, 1)
      wait_out(t, bo - 1)
      transpose(t)
      fire_out(t, bo)

    return carry

  lax.fori_loop(0, BPW // 2, j_loop, 0)

  blast = b0 + BPW - 1
  wait_gather(0, 0)
  for t in range(NT):
    wait_out(t, blast)


def kernel(p, a, f, p_emb, a_emb, f_emb):
  pr = p.astype(jnp.int32)
  ar = a.astype(jnp.int32)
  fr = f.astype(jnp.int32)
  k = pl.kernel(
      _body,
      out_type=jax.ShapeDtypeStruct((B * NT * BLK,), jnp.float32),
      mesh=plsc.VectorSubcoreMesh(core_axis_name="c", subcore_axis_name="s"),
      compiler_params=pltpu.CompilerParams(needs_layout_passes=False,
                                           use_tc_tiling_on_sc=False),
      scratch_types=(
          [pltpu.VMEM((2 * NT, GC), jnp.int32) for _ in range(2)]
          + [pltpu.VMEM((LP, C), jnp.float32) for _ in range(NT)]
          + [pltpu.VMEM((SBUF,), jnp.float32)]
          + [pltpu.VMEM((BLK + 8,), jnp.float32) for _ in range(NT)]
          + [pltpu.SemaphoreType.DMA for _ in range(2 + NT + NT)]
      ),
  )
  out = k(pr, ar, fr, p_emb, a_emb, f_emb)
  return out.reshape(B, NT * C, L)


# speedup vs baseline: 1.1047x; 1.1047x over previous
import functools

import jax
import jax.numpy as jnp
from jax import lax
from jax.experimental import pallas as pl
from jax.experimental.pallas import tpu as pltpu
from jax.experimental.pallas import tpu_sc as plsc

B = 4096
L = 200
C = 64
NT = 3
NC, NS = 2, 16
NW = NC * NS
BPW = B // NW
LP = 208
GC = 104
G2 = 96
SCALE = 8.0


def _body(p_idx, a_idx, f_idx, p_emb, a_emb, f_emb, out_ref,
          i_s0, i_s1, r_v0, r_v1, r_v2,
          is0, is1, gs0, gs1, gs2, os0, os1, os2):
  islab = [i_s0, i_s1]
  rows_v = [r_v0, r_v1, r_v2]
  isem = [is0, is1]
  gsem = [gs0, gs1, gs2]
  osem = [os0, os1, os2]
  tabs = [p_emb, a_emb, f_emb]
  idx_hbm = [p_idx, a_idx, f_idx]

  wid = lax.axis_index("s") * NC + lax.axis_index("c")
  b0 = wid * BPW

  def fire_idx(par, b):
    for t in range(NT):
      pltpu.async_copy(idx_hbm[t].at[b, pl.ds(0, GC)],
                       islab[par].at[2 * t], isem[par])
      pltpu.async_copy(idx_hbm[t].at[b, pl.ds(G2, GC)],
                       islab[par].at[2 * t + 1], isem[par])

  def wait_idx(par, b):
    for t in range(NT):
      pltpu.make_async_copy(idx_hbm[t].at[b, pl.ds(0, GC)],
                            islab[par].at[2 * t], isem[par]).wait()
      pltpu.make_async_copy(idx_hbm[t].at[b, pl.ds(G2, GC)],
                            islab[par].at[2 * t + 1], isem[par]).wait()

  def fire_gather(t, par):
    pltpu.async_copy(tabs[t].at[islab[par].at[2 * t]],
                     rows_v[t].at[pl.ds(0, GC)], gsem[t])
    pltpu.async_copy(tabs[t].at[islab[par].at[2 * t + 1]],
                     rows_v[t].at[pl.ds(G2, GC)], gsem[t])

  def wait_gather(t, par):
    pltpu.make_async_copy(tabs[t].at[islab[par].at[2 * t]],
                          rows_v[t].at[pl.ds(0, GC)], gsem[t]).wait()
    pltpu.make_async_copy(tabs[t].at[islab[par].at[2 * t + 1]],
                          rows_v[t].at[pl.ds(G2, GC)], gsem[t]).wait()

  def fire_out(t, b):
    pltpu.async_copy(rows_v[t].at[pl.ds(0, L)],
                     out_ref.at[b, pl.ds(0, L), pl.ds(t * C, C)], osem[t])

  def wait_out(t, b):
    pltpu.make_async_copy(rows_v[t].at[pl.ds(0, L)],
                          out_ref.at[b, pl.ds(0, L), pl.ds(t * C, C)],
                          osem[t]).wait()

  fire_idx(0, b0)
  wait_idx(0, b0)
  fire_gather(0, 0)

  def j_loop(j, carry):
    b = b0 + 2 * j
    bo = b + 1
    bn = b0 + jnp.minimum(2 * j + 2, BPW - 1)

    for t in range(NT):
      if t == 0:
        fire_idx(1, bo)
      if t < NT - 1:

        @pl.when(j > 0)
        def _():
          wait_out(t + 1, bo - 2)

        fire_gather(t + 1, 0)
      else:
        wait_idx(1, bo)
        wait_out(0, b)
        fire_gather(0, 1)
      wait_gather(t, 0)
      fire_out(t, b)

    for t in range(NT):
      if t == 0:
        fire_idx(0, bn)
      if t < NT - 1:
        wait_out(t + 1, b)
        fire_gather(t + 1, 1)
      else:
        wait_idx(0, bn)
        wait_out(0, bo)
        fire_gather(0, 0)
      wait_gather(t, 1)
      fire_out(t, bo)

    return carry

  lax.fori_loop(0, BPW // 2, j_loop, 0)

  blast = b0 + BPW - 1
  wait_gather(0, 0)
  for t in (1, 2):
    wait_out(t, blast)


def kernel(p, a, f, p_emb, a_emb, f_emb):
  pr = p.astype(jnp.int32)
  ar = a.astype(jnp.int32)
  fr = f.astype(jnp.int32)
  k = pl.kernel(
      _body,
      out_type=jax.ShapeDtypeStruct((B, L, NT * C), jnp.float32),
      mesh=plsc.VectorSubcoreMesh(core_axis_name="c", subcore_axis_name="s"),
      compiler_params=pltpu.CompilerParams(needs_layout_passes=False,
                                           use_tc_tiling_on_sc=False),
      scratch_types=(
          [pltpu.VMEM((2 * NT, GC), jnp.int32) for _ in range(2)]
          + [pltpu.VMEM((LP, C), jnp.float32) for _ in range(NT)]
          + [pltpu.SemaphoreType.DMA for _ in range(2 + NT + NT)]
      ),
  )
  out = k(pr, ar, fr, p_emb, a_emb, f_emb)
  return jnp.swapaxes(out, -1, -2) * SCALE
